# Initial kernel scaffold; baseline (speedup 1.0000x reference)
import jax
import jax.numpy as jnp
from jax.experimental import pallas as pl


def kernel(poi, demo, image, mob_adj, global_indices, ebds, W_poi, b_poi, W1, b1, W2, b2, W_img, b_img):
    raise NotImplementedError("write your pallas kernel here")


# speedup vs baseline: 1.0652x; 1.0652x over previous
import functools

import jax
import jax.numpy as jnp
from jax import lax
from jax.experimental import pallas as pl
from jax.experimental.pallas import tpu as pltpu
from jax.experimental.pallas import tpu_sc as plsc

N = 10000
D = 128
B = 4096
SCALE = 1.0 / 0.07
K0 = 9984
KTAIL = N - K0

_NC, _NS = 2, 16
_NW = _NC * _NS
_RPW = B // _NW
_CK = 8


def _sc_gather_adj_body(adj_hbm, gi_hbm, g_hbm, idx_v, rowbuf, sem):
    wid = lax.axis_index("s") * _NC + lax.axis_index("c")
    base = wid * _RPW
    pltpu.sync_copy(gi_hbm.at[pl.ds(base, _RPW)], idx_v)

    def body(j, carry):
        pltpu.async_copy(
            adj_hbm.at[idx_v.at[pl.ds(j * _CK, _CK)], pl.ds(0, K0)],
            rowbuf, sem).wait()
        pltpu.sync_copy(rowbuf, g_hbm.at[pl.ds(base + j * _CK, _CK)])
        return carry

    lax.fori_loop(0, _RPW // _CK, body, 0)


def _sc_gather_rows_body(e2_hbm, gi_hbm, eg_hbm, idx_v, ebuf, sem):
    wid = lax.axis_index("s") * _NC + lax.axis_index("c")
    base = wid * _RPW
    pltpu.sync_copy(gi_hbm.at[pl.ds(base, _RPW)], idx_v)
    pltpu.async_copy(e2_hbm.at[idx_v], ebuf, sem).wait()
    pltpu.sync_copy(ebuf, eg_hbm.at[pl.ds(base, _RPW)])


def _sc_mesh():
    return plsc.VectorSubcoreMesh(core_axis_name="c", subcore_axis_name="s",
                                  num_cores=_NC, num_subcores=_NS)


@functools.cache
def _sc_gather_adj_kernel():
    return pl.kernel(
        _sc_gather_adj_body,
        out_type=jax.ShapeDtypeStruct((B, K0), jnp.float32),
        mesh=_sc_mesh(),
        scratch_types=[
            pltpu.VMEM((_RPW,), jnp.int32),
            pltpu.VMEM((_CK, K0), jnp.float32),
            pltpu.SemaphoreType.DMA,
        ],
    )


@functools.cache
def _sc_gather_rows_kernel():
    return pl.kernel(
        _sc_gather_rows_body,
        out_type=jax.ShapeDtypeStruct((B, D), jnp.float32),
        mesh=_sc_mesh(),
        scratch_types=[
            pltpu.VMEM((_RPW,), jnp.int32),
            pltpu.VMEM((_RPW, D), jnp.float32),
            pltpu.SemaphoreType.DMA,
        ],
    )


def _ytail_body(a_ref, ef_ref, er_ref, y_ref):
    y_ref[...] = er_ref[...] + jnp.dot(
        a_ref[...], ef_ref[...], preferred_element_type=jnp.float32)


def _ytail(adj, ebds):
    return pl.pallas_call(
        _ytail_body,
        grid=(1,),
        in_specs=[
            pl.BlockSpec((KTAIL, N), lambda i: (N // KTAIL - 1, 0)),
            pl.BlockSpec((N, D), lambda i: (0, 0)),
            pl.BlockSpec((KTAIL, D), lambda i: (N // KTAIL - 1, 0)),
        ],
        out_specs=pl.BlockSpec((KTAIL, D), lambda i: (0, 0)),
        out_shape=jax.ShapeDtypeStruct((KTAIL, D), jnp.float32),
    )(adj, ebds, ebds)


_BM_E2 = 512


def _e2_body(atail_ref, yt_ref, e_ref, e2_ref):
    at = atail_ref[:, :KTAIL]
    e2_ref[...] = e_ref[...] + jnp.dot(
        at, yt_ref[...], preferred_element_type=jnp.float32)


def _e2(adj, ytail, ebds):
    nblk = pl.cdiv(N, _BM_E2)
    nlane = pl.cdiv(N, 128)
    return pl.pallas_call(
        _e2_body,
        grid=(nblk,),
        in_specs=[
            pl.BlockSpec((_BM_E2, 128), lambda i: (i, nlane - 1)),
            pl.BlockSpec((KTAIL, D), lambda i: (0, 0)),
            pl.BlockSpec((_BM_E2, D), lambda i: (i, 0)),
        ],
        out_specs=pl.BlockSpec((_BM_E2, D), lambda i: (i, 0)),
        out_shape=jax.ShapeDtypeStruct((N, D), jnp.float32),
        compiler_params=pltpu.CompilerParams(
            dimension_semantics=("arbitrary",)),
    )(adj, ytail, ebds)


_BM_PROP = 256


def _prop_body(a_ref, ef_ref, er_ref, y_ref):
    y_ref[...] = er_ref[...] + jnp.dot(
        a_ref[...], ef_ref[...], preferred_element_type=jnp.float32)


def _propagate(adj, ebds):
    nblk = K0 // _BM_PROP
    return pl.pallas_call(
        _prop_body,
        grid=(nblk,),
        in_specs=[
            pl.BlockSpec((_BM_PROP, N), lambda i: (i, 0)),
            pl.BlockSpec((N, D), lambda i: (0, 0)),
            pl.BlockSpec((_BM_PROP, D), lambda i: (i, 0)),
        ],
        out_specs=pl.BlockSpec((_BM_PROP, D), lambda i: (i, 0)),
        out_shape=jax.ShapeDtypeStruct((K0, D), jnp.float32),
        compiler_params=pltpu.CompilerParams(
            dimension_semantics=("arbitrary",)),
    )(adj, ebds, ebds)


_BM_MOB = 256


def _mob_body(g_ref, y_ref, eg_ref, m_ref, mn_ref):
    m = eg_ref[...] + jnp.dot(
        g_ref[...], y_ref[...], preferred_element_type=jnp.float32)
    m_ref[...] = m
    mn_ref[...] = m / jnp.sqrt(jnp.sum(m * m, axis=1, keepdims=True))


def _mob_embed(g, y99, eg2):
    nblk = B // _BM_MOB
    return pl.pallas_call(
        _mob_body,
        grid=(nblk,),
        in_specs=[
            pl.BlockSpec((_BM_MOB, K0), lambda i: (i, 0)),
            pl.BlockSpec((K0, D), lambda i: (0, 0)),
            pl.BlockSpec((_BM_MOB, D), lambda i: (i, 0)),
        ],
        out_specs=[
            pl.BlockSpec((_BM_MOB, D), lambda i: (i, 0)),
            pl.BlockSpec((_BM_MOB, D), lambda i: (i, 0)),
        ],
        out_shape=[
            jax.ShapeDtypeStruct((B, D), jnp.float32),
            jax.ShapeDtypeStruct((B, D), jnp.float32),
        ],
        compiler_params=pltpu.CompilerParams(
            dimension_semantics=("arbitrary",)),
    )(g, y99, eg2)


_BT = 512


def _towers_body(poi_ref, demo_ref, img_ref, wp_ref, bp_ref, w1_ref, b1_ref,
                 w2_ref, b2_ref, wi_ref, bi_ref, pn_ref, dn_ref, in_ref):
    def norm(x):
        return x / jnp.sqrt(jnp.sum(x * x, axis=1, keepdims=True))

    p = jnp.dot(poi_ref[...], wp_ref[...],
                preferred_element_type=jnp.float32) + bp_ref[...]
    pn_ref[...] = norm(p)
    h = jnp.maximum(jnp.dot(demo_ref[...], w1_ref[...],
                            preferred_element_type=jnp.float32) + b1_ref[...],
                    0.0)
    dd = jnp.dot(h, w2_ref[...],
                 preferred_element_type=jnp.float32) + b2_ref[...]
    dn_ref[...] = norm(dd)
    im = jnp.dot(img_ref[...], wi_ref[...],
                 preferred_element_type=jnp.float32) + bi_ref[...]
    in_ref[...] = norm(im)


def _towers(poi, demo, image, W_poi, b_poi, W1, b1, W2, b2, W_img, b_img):
    nblk = B // _BT
    poi_d, demo_d, img_d, demo_h = (W_poi.shape[0], W1.shape[0],
                                    W_img.shape[0], W1.shape[1])
    full = lambda shape: pl.BlockSpec(shape, lambda i: tuple(0 for _ in shape))
    return pl.pallas_call(
        _towers_body,
        grid=(nblk,),
        in_specs=[
            pl.BlockSpec((_BT, poi_d), lambda i: (i, 0)),
            pl.BlockSpec((_BT, demo_d), lambda i: (i, 0)),
            pl.BlockSpec((_BT, img_d), lambda i: (i, 0)),
            full((poi_d, D)), full((1, D)),
            full((demo_d, demo_h)), full((1, demo_h)),
            full((demo_h, D)), full((1, D)),
            full((img_d, D)), full((1, D)),
        ],
        out_specs=[pl.BlockSpec((_BT, D), lambda i: (i, 0))] * 3,
        out_shape=[jax.ShapeDtypeStruct((B, D), jnp.float32)] * 3,
        compiler_params=pltpu.CompilerParams(
            dimension_semantics=("arbitrary",)),
    )(poi, demo, image, W_poi, b_poi.reshape(1, -1), W1, b1.reshape(1, -1),
      W2, b2.reshape(1, -1), W_img, b_img.reshape(1, -1))


_BL = 512


def _logits_body(mn_ref, pn_ref, dn_ref, in_ref,
                 lmp_ref, lpm_ref, lmd_ref, ldm_ref, lmi_ref, lim_ref):
    def dg(a, b):
        return SCALE * lax.dot_general(
            a, b, (((1,), (1,)), ((), ())),
            preferred_element_type=jnp.float32)

    m = mn_ref[...]
    p, d, i = pn_ref[...], dn_ref[...], in_ref[...]
    lmp_ref[...] = dg(m, p)
    lpm_ref[...] = dg(p, m)
    lmd_ref[...] = dg(m, d)
    ldm_ref[...] = dg(d, m)
    lmi_ref[...] = dg(m, i)
    lim_ref[...] = dg(i, m)


def _logits(mn, pn, dn, imn):
    nblk = B // _BL
    row = pl.BlockSpec((_BL, D), lambda i, j: (i, 0))
    col = pl.BlockSpec((_BL, D), lambda i, j: (j, 0))
    out_ij = pl.BlockSpec((_BL, _BL), lambda i, j: (i, j))
    out_ji = pl.BlockSpec((_BL, _BL), lambda i, j: (j, i))
    ls = jax.ShapeDtypeStruct((B, B), jnp.float32)
    return pl.pallas_call(
        _logits_body,
        grid=(nblk, nblk),
        in_specs=[row, col, col, col],
        out_specs=[out_ij, out_ji, out_ij, out_ji, out_ij, out_ji],
        out_shape=[ls] * 6,
        compiler_params=pltpu.CompilerParams(
            dimension_semantics=("parallel", "parallel")),
    )(mn, pn, dn, imn)


def kernel(poi, demo, image, mob_adj, global_indices, ebds,
           W_poi, b_poi, W1, b1, W2, b2, W_img, b_img):
    g = _sc_gather_adj_kernel()(mob_adj, global_indices)
    ytail = _ytail(mob_adj, ebds)
    e2 = _e2(mob_adj, ytail, ebds)
    y99 = _propagate(mob_adj, ebds)
    eg2 = _sc_gather_rows_kernel()(e2, global_indices)
    mob_ebd, mob_n = _mob_embed(g, y99, eg2)
    poi_n, demo_n, img_n = _towers(poi, demo, image, W_poi, b_poi,
                                   W1, b1, W2, b2, W_img, b_img)
    l_mp, l_pm, l_md, l_dm, l_mi, l_im = _logits(mob_n, poi_n, demo_n, img_n)
    return (l_mp, l_pm, l_md, l_dm, l_mi, l_im, mob_ebd)


# speedup vs baseline: 1.2921x; 1.2130x over previous
import functools

import jax
import jax.numpy as jnp
from jax import lax
from jax.experimental import pallas as pl
from jax.experimental.pallas import tpu as pltpu
from jax.experimental.pallas import tpu_sc as plsc

N = 10000
D = 128
B = 4096
SCALE = 1.0 / 0.07

_NC, _NS = 2, 16
_NW = _NC * _NS
_RPW = B // _NW


def _sc_gather_rows_body(tab_hbm, gi_hbm, out_hbm, idx_v, ebuf, sem):
    wid = lax.axis_index("s") * _NC + lax.axis_index("c")
    base = wid * _RPW
    pltpu.sync_copy(gi_hbm.at[pl.ds(base, _RPW)], idx_v)
    pltpu.async_copy(tab_hbm.at[idx_v], ebuf, sem).wait()
    pltpu.sync_copy(ebuf, out_hbm.at[pl.ds(base, _RPW)])


@functools.cache
def _sc_gather_rows_kernel():
    return pl.kernel(
        _sc_gather_rows_body,
        out_type=jax.ShapeDtypeStruct((B, D), jnp.float32),
        mesh=plsc.VectorSubcoreMesh(core_axis_name="c", subcore_axis_name="s",
                                    num_cores=_NC, num_subcores=_NS),
        scratch_types=[
            pltpu.VMEM((_RPW,), jnp.int32),
            pltpu.VMEM((_RPW, D), jnp.float32),
            pltpu.SemaphoreType.DMA,
        ],
    )


_BM_PROP = 400


def _prop_body(a_ref, ef_ref, er_ref, y_ref):
    y_ref[...] = er_ref[...] + jnp.dot(
        a_ref[...], ef_ref[...], preferred_element_type=jnp.float32)


def _propagate(adj, ebds):
    nblk = N // _BM_PROP
    return pl.pallas_call(
        _prop_body,
        grid=(nblk,),
        in_specs=[
            pl.BlockSpec((_BM_PROP, N), lambda i: (i, 0)),
            pl.BlockSpec((N, D), lambda i: (0, 0)),
            pl.BlockSpec((_BM_PROP, D), lambda i: (i, 0)),
        ],
        out_specs=pl.BlockSpec((_BM_PROP, D), lambda i: (i, 0)),
        out_shape=jax.ShapeDtypeStruct((N, D), jnp.float32),
        compiler_params=pltpu.CompilerParams(
            dimension_semantics=("arbitrary",)),
    )(adj, ebds, ebds)


_BM_MOB = 256


def _mob_body(gi_ref, adj_ref, y_ref, eg_ref, m_ref, mn_ref,
              buf0, buf1, sem0, sem1):
    i = pl.program_id(0)
    nblk = pl.num_programs(0)

    def dma(b, blk, buf, sem):
        row = gi_ref[blk * _BM_MOB + b]
        return pltpu.make_async_copy(
            adj_ref.at[pl.ds(row, 1)], buf.at[pl.ds(b, 1)], sem)

    def issue(blk, buf, sem):
        def f(b, c):
            dma(b, blk, buf, sem).start()
            return c
        lax.fori_loop(0, _BM_MOB, f, 0)

    def drain(blk, buf, sem):
        def f(b, c):
            dma(b, blk, buf, sem).wait()
            return c
        lax.fori_loop(0, _BM_MOB, f, 0)

    @pl.when(i == 0)
    def _():
        issue(0, buf0, sem0)

    even = i % 2 == 0

    @pl.when((i + 1 < nblk) & even)
    def _():
        issue(i + 1, buf1, sem1)

    @pl.when((i + 1 < nblk) & jnp.logical_not(even))
    def _():
        issue(i + 1, buf0, sem0)

    def compute(buf, sem):
        drain(i, buf, sem)
        m = eg_ref[...] + jnp.dot(
            buf[...], y_ref[...], preferred_element_type=jnp.float32)
        m_ref[...] = m
        mn_ref[...] = m / jnp.sqrt(jnp.sum(m * m, axis=1, keepdims=True))

    @pl.when(even)
    def _():
        compute(buf0, sem0)

    @pl.when(jnp.logical_not(even))
    def _():
        compute(buf1, sem1)


def _mob_embed(gi, adj, y, eg):
    nblk = B // _BM_MOB
    grid_spec = pltpu.PrefetchScalarGridSpec(
        num_scalar_prefetch=1,
        grid=(nblk,),
        in_specs=[
            pl.BlockSpec(memory_space=pl.ANY),
            pl.BlockSpec((N, D), lambda i, gi_ref: (0, 0)),
            pl.BlockSpec((_BM_MOB, D), lambda i, gi_ref: (i, 0)),
        ],
        out_specs=[
            pl.BlockSpec((_BM_MOB, D), lambda i, gi_ref: (i, 0)),
            pl.BlockSpec((_BM_MOB, D), lambda i, gi_ref: (i, 0)),
        ],
        scratch_shapes=[
            pltpu.VMEM((_BM_MOB, N), jnp.float32),
            pltpu.VMEM((_BM_MOB, N), jnp.float32),
            pltpu.SemaphoreType.DMA,
            pltpu.SemaphoreType.DMA,
        ],
    )
    return pl.pallas_call(
        _mob_body,
        grid_spec=grid_spec,
        out_shape=[
            jax.ShapeDtypeStruct((B, D), jnp.float32),
            jax.ShapeDtypeStruct((B, D), jnp.float32),
        ],
        compiler_params=pltpu.CompilerParams(
            dimension_semantics=("arbitrary",)),
    )(gi, adj, y, eg)


_BT = 512


def _towers_body(poi_ref, demo_ref, img_ref, wp_ref, bp_ref, w1_ref, b1_ref,
                 w2_ref, b2_ref, wi_ref, bi_ref, pn_ref, dn_ref, in_ref):
    def norm(x):
        return x / jnp.sqrt(jnp.sum(x * x, axis=1, keepdims=True))

    p = jnp.dot(poi_ref[...], wp_ref[...],
                preferred_element_type=jnp.float32) + bp_ref[...]
    pn_ref[...] = norm(p)
    h = jnp.maximum(jnp.dot(demo_ref[...], w1_ref[...],
                            preferred_element_type=jnp.float32) + b1_ref[...],
                    0.0)
    dd = jnp.dot(h, w2_ref[...],
                 preferred_element_type=jnp.float32) + b2_ref[...]
    dn_ref[...] = norm(dd)
    im = jnp.dot(img_ref[...], wi_ref[...],
                 preferred_element_type=jnp.float32) + bi_ref[...]
    in_ref[...] = norm(im)


def _towers(poi, demo, image, W_poi, b_poi, W1, b1, W2, b2, W_img, b_img):
    nblk = B // _BT
    poi_d, demo_d, img_d, demo_h = (W_poi.shape[0], W1.shape[0],
                                    W_img.shape[0], W1.shape[1])
    full = lambda shape: pl.BlockSpec(shape, lambda i: tuple(0 for _ in shape))
    return pl.pallas_call(
        _towers_body,
        grid=(nblk,),
        in_specs=[
            pl.BlockSpec((_BT, poi_d), lambda i: (i, 0)),
            pl.BlockSpec((_BT, demo_d), lambda i: (i, 0)),
            pl.BlockSpec((_BT, img_d), lambda i: (i, 0)),
            full((poi_d, D)), full((1, D)),
            full((demo_d, demo_h)), full((1, demo_h)),
            full((demo_h, D)), full((1, D)),
            full((img_d, D)), full((1, D)),
        ],
        out_specs=[pl.BlockSpec((_BT, D), lambda i: (i, 0))] * 3,
        out_shape=[jax.ShapeDtypeStruct((B, D), jnp.float32)] * 3,
        compiler_params=pltpu.CompilerParams(
            dimension_semantics=("arbitrary",)),
    )(poi, demo, image, W_poi, b_poi.reshape(1, -1), W1, b1.reshape(1, -1),
      W2, b2.reshape(1, -1), W_img, b_img.reshape(1, -1))


_BL = 512


def _logits_body(mn_ref, pn_ref, dn_ref, in_ref,
                 lmp_ref, lpm_ref, lmd_ref, ldm_ref, lmi_ref, lim_ref):
    def dg(a, b):
        return SCALE * lax.dot_general(
            a, b, (((1,), (1,)), ((), ())),
            preferred_element_type=jnp.float32)

    m = mn_ref[...]
    p, d, i = pn_ref[...], dn_ref[...], in_ref[...]
    lmp_ref[...] = dg(m, p)
    lpm_ref[...] = dg(p, m)
    lmd_ref[...] = dg(m, d)
    ldm_ref[...] = dg(d, m)
    lmi_ref[...] = dg(m, i)
    lim_ref[...] = dg(i, m)


def _logits(mn, pn, dn, imn):
    nblk = B // _BL
    row = pl.BlockSpec((_BL, D), lambda i, j: (i, 0))
    col = pl.BlockSpec((_BL, D), lambda i, j: (j, 0))
    out_ij = pl.BlockSpec((_BL, _BL), lambda i, j: (i, j))
    out_ji = pl.BlockSpec((_BL, _BL), lambda i, j: (j, i))
    ls = jax.ShapeDtypeStruct((B, B), jnp.float32)
    return pl.pallas_call(
        _logits_body,
        grid=(nblk, nblk),
        in_specs=[row, col, col, col],
        out_specs=[out_ij, out_ji, out_ij, out_ji, out_ij, out_ji],
        out_shape=[ls] * 6,
        compiler_params=pltpu.CompilerParams(
            dimension_semantics=("parallel", "parallel")),
    )(mn, pn, dn, imn)


def kernel(poi, demo, image, mob_adj, global_indices, ebds,
           W_poi, b_poi, W1, b1, W2, b2, W_img, b_img):
    eg = _sc_gather_rows_kernel()(ebds, global_indices)
    y = _propagate(mob_adj, ebds)
    mob_ebd, mob_n = _mob_embed(global_indices, mob_adj, y, eg)
    poi_n, demo_n, img_n = _towers(poi, demo, image, W_poi, b_poi,
                                   W1, b1, W2, b2, W_img, b_img)
    l_mp, l_pm, l_md, l_dm, l_mi, l_im = _logits(mob_n, poi_n, demo_n, img_n)
    return (l_mp, l_pm, l_md, l_dm, l_mi, l_im, mob_ebd)


# speedup vs baseline: 1.3594x; 1.0521x over previous
import functools

import jax
import jax.numpy as jnp
from jax import lax
from jax.experimental import pallas as pl
from jax.experimental.pallas import tpu as pltpu
from jax.experimental.pallas import tpu_sc as plsc

N = 10000
D = 128
B = 4096
SCALE = 1.0 / 0.07

_NC, _NS = 2, 16
_NW = _NC * _NS
_RPW = B // _NW


def _sc_gather_rows_body(tab_hbm, gi_hbm, out_hbm, idx_v, ebuf, sem):
    wid = lax.axis_index("s") * _NC + lax.axis_index("c")
    base = wid * _RPW
    pltpu.sync_copy(gi_hbm.at[pl.ds(base, _RPW)], idx_v)
    pltpu.async_copy(tab_hbm.at[idx_v], ebuf, sem).wait()
    pltpu.sync_copy(ebuf, out_hbm.at[pl.ds(base, _RPW)])


@functools.cache
def _sc_gather_rows_kernel():
    return pl.kernel(
        _sc_gather_rows_body,
        out_type=jax.ShapeDtypeStruct((B, D), jnp.float32),
        mesh=plsc.VectorSubcoreMesh(core_axis_name="c", subcore_axis_name="s",
                                    num_cores=_NC, num_subcores=_NS),
        scratch_types=[
            pltpu.VMEM((_RPW,), jnp.int32),
            pltpu.VMEM((_RPW, D), jnp.float32),
            pltpu.SemaphoreType.DMA,
        ],
    )


_BM_PROP = 400


def _prop_body(a_ref, ef_ref, er_ref, y_ref):
    y_ref[...] = er_ref[...] + jnp.dot(
        a_ref[...], ef_ref[...], preferred_element_type=jnp.float32)


def _propagate(adj, ebds):
    nblk = N // _BM_PROP
    return pl.pallas_call(
        _prop_body,
        grid=(nblk,),
        in_specs=[
            pl.BlockSpec((_BM_PROP, N), lambda i: (i, 0)),
            pl.BlockSpec((N, D), lambda i: (0, 0)),
            pl.BlockSpec((_BM_PROP, D), lambda i: (i, 0)),
        ],
        out_specs=pl.BlockSpec((_BM_PROP, D), lambda i: (i, 0)),
        out_shape=jax.ShapeDtypeStruct((N, D), jnp.float32),
        compiler_params=pltpu.CompilerParams(
            dimension_semantics=("arbitrary",)),
    )(adj, ebds, ebds)


_BM_MOB = 256


def _mob_body(gi_ref, adj_ref, y_ref, eg_ref, m_ref, mn_ref,
              buf0, buf1, sem0, sem1):
    i = pl.program_id(0)
    nblk = pl.num_programs(0)

    def dma(b, blk, buf, sem):
        row = gi_ref[blk * _BM_MOB + b]
        return pltpu.make_async_copy(
            adj_ref.at[pl.ds(row, 1)], buf.at[pl.ds(b, 1)], sem)

    def issue(blk, buf, sem):
        def f(b, c):
            dma(b, blk, buf, sem).start()
            return c
        lax.fori_loop(0, _BM_MOB, f, 0)

    def drain(blk, buf, sem):
        def f(b, c):
            dma(b, blk, buf, sem).wait()
            return c
        lax.fori_loop(0, _BM_MOB, f, 0)

    @pl.when(i == 0)
    def _():
        issue(0, buf0, sem0)

    even = i % 2 == 0

    @pl.when((i + 1 < nblk) & even)
    def _():
        issue(i + 1, buf1, sem1)

    @pl.when((i + 1 < nblk) & jnp.logical_not(even))
    def _():
        issue(i + 1, buf0, sem0)

    def compute(buf, sem):
        drain(i, buf, sem)
        m = eg_ref[...] + jnp.dot(
            buf[...], y_ref[...], preferred_element_type=jnp.float32)
        m_ref[...] = m
        mn_ref[...] = m / jnp.sqrt(jnp.sum(m * m, axis=1, keepdims=True))

    @pl.when(even)
    def _():
        compute(buf0, sem0)

    @pl.when(jnp.logical_not(even))
    def _():
        compute(buf1, sem1)


def _mob_embed(gi, adj, y, eg):
    nblk = B // _BM_MOB
    grid_spec = pltpu.PrefetchScalarGridSpec(
        num_scalar_prefetch=1,
        grid=(nblk,),
        in_specs=[
            pl.BlockSpec(memory_space=pl.ANY),
            pl.BlockSpec((N, D), lambda i, gi_ref: (0, 0)),
            pl.BlockSpec((_BM_MOB, D), lambda i, gi_ref: (i, 0)),
        ],
        out_specs=[
            pl.BlockSpec((_BM_MOB, D), lambda i, gi_ref: (i, 0)),
            pl.BlockSpec((_BM_MOB, D), lambda i, gi_ref: (i, 0)),
        ],
        scratch_shapes=[
            pltpu.VMEM((_BM_MOB, N), jnp.float32),
            pltpu.VMEM((_BM_MOB, N), jnp.float32),
            pltpu.SemaphoreType.DMA,
            pltpu.SemaphoreType.DMA,
        ],
    )
    return pl.pallas_call(
        _mob_body,
        grid_spec=grid_spec,
        out_shape=[
            jax.ShapeDtypeStruct((B, D), jnp.float32),
            jax.ShapeDtypeStruct((B, D), jnp.float32),
        ],
        compiler_params=pltpu.CompilerParams(
            dimension_semantics=("arbitrary",)),
    )(gi, adj, y, eg)


_BT = 512


def _towers_body(poi_ref, demo_ref, img_ref, wp_ref, bp_ref, w1_ref, b1_ref,
                 w2_ref, b2_ref, wi_ref, bi_ref, pn_ref, dn_ref, in_ref):
    def norm(x):
        return x / jnp.sqrt(jnp.sum(x * x, axis=1, keepdims=True))

    p = jnp.dot(poi_ref[...], wp_ref[...],
                preferred_element_type=jnp.float32) + bp_ref[...]
    pn_ref[...] = norm(p)
    h = jnp.maximum(jnp.dot(demo_ref[...], w1_ref[...],
                            preferred_element_type=jnp.float32) + b1_ref[...],
                    0.0)
    dd = jnp.dot(h, w2_ref[...],
                 preferred_element_type=jnp.float32) + b2_ref[...]
    dn_ref[...] = norm(dd)
    im = jnp.dot(img_ref[...], wi_ref[...],
                 preferred_element_type=jnp.float32) + bi_ref[...]
    in_ref[...] = norm(im)


def _towers(poi, demo, image, W_poi, b_poi, W1, b1, W2, b2, W_img, b_img):
    nblk = B // _BT
    poi_d, demo_d, img_d, demo_h = (W_poi.shape[0], W1.shape[0],
                                    W_img.shape[0], W1.shape[1])
    full = lambda shape: pl.BlockSpec(shape, lambda i: tuple(0 for _ in shape))
    return pl.pallas_call(
        _towers_body,
        grid=(nblk,),
        in_specs=[
            pl.BlockSpec((_BT, poi_d), lambda i: (i, 0)),
            pl.BlockSpec((_BT, demo_d), lambda i: (i, 0)),
            pl.BlockSpec((_BT, img_d), lambda i: (i, 0)),
            full((poi_d, D)), full((1, D)),
            full((demo_d, demo_h)), full((1, demo_h)),
            full((demo_h, D)), full((1, D)),
            full((img_d, D)), full((1, D)),
        ],
        out_specs=[pl.BlockSpec((_BT, D), lambda i: (i, 0))] * 3,
        out_shape=[jax.ShapeDtypeStruct((B, D), jnp.float32)] * 3,
        compiler_params=pltpu.CompilerParams(
            dimension_semantics=("arbitrary",)),
    )(poi, demo, image, W_poi, b_poi.reshape(1, -1), W1, b1.reshape(1, -1),
      W2, b2.reshape(1, -1), W_img, b_img.reshape(1, -1))


_BL = 512


def _logits_body(mn_ref, pn_ref, dn_ref, in_ref,
                 lmp_ref, lpm_ref, lmd_ref, ldm_ref, lmi_ref, lim_ref):
    def dg(a, b):
        return SCALE * lax.dot_general(
            a, b, (((1,), (1,)), ((), ())),
            preferred_element_type=jnp.float32)

    bi = pl.program_id(0) * _BL
    bj = pl.program_id(1) * _BL
    m = mn_ref[pl.ds(bi, _BL), :]
    p = pn_ref[pl.ds(bj, _BL), :]
    d = dn_ref[pl.ds(bj, _BL), :]
    i = in_ref[pl.ds(bj, _BL), :]
    lmp_ref[...] = dg(m, p)
    lpm_ref[...] = dg(p, m)
    lmd_ref[...] = dg(m, d)
    ldm_ref[...] = dg(d, m)
    lmi_ref[...] = dg(m, i)
    lim_ref[...] = dg(i, m)


def _logits(mn, pn, dn, imn):
    nblk = B // _BL
    full = pl.BlockSpec((B, D), lambda i, j: (0, 0))
    out_ij = pl.BlockSpec((_BL, _BL), lambda i, j: (i, j))
    out_ji = pl.BlockSpec((_BL, _BL), lambda i, j: (j, i))
    ls = jax.ShapeDtypeStruct((B, B), jnp.float32)
    return pl.pallas_call(
        _logits_body,
        grid=(nblk, nblk),
        in_specs=[full, full, full, full],
        out_specs=[out_ij, out_ji, out_ij, out_ji, out_ij, out_ji],
        out_shape=[ls] * 6,
        compiler_params=pltpu.CompilerParams(
            dimension_semantics=("parallel", "parallel")),
    )(mn, pn, dn, imn)


def kernel(poi, demo, image, mob_adj, global_indices, ebds,
           W_poi, b_poi, W1, b1, W2, b2, W_img, b_img):
    eg = _sc_gather_rows_kernel()(ebds, global_indices)
    y = _propagate(mob_adj, ebds)
    mob_ebd, mob_n = _mob_embed(global_indices, mob_adj, y, eg)
    poi_n, demo_n, img_n = _towers(poi, demo, image, W_poi, b_poi,
                                   W1, b1, W2, b2, W_img, b_img)
    l_mp, l_pm, l_md, l_dm, l_mi, l_im = _logits(mob_n, poi_n, demo_n, img_n)
    return (l_mp, l_pm, l_md, l_dm, l_mi, l_im, mob_ebd)


# speedup vs baseline: 1.3697x; 1.0076x over previous
import functools

import jax
import jax.numpy as jnp
from jax import lax
from jax.experimental import pallas as pl
from jax.experimental.pallas import tpu as pltpu
from jax.experimental.pallas import tpu_sc as plsc

N = 10000
D = 128
B = 4096
SCALE = 1.0 / 0.07

_NC, _NS = 2, 16
_NW = _NC * _NS
_RPW = B // _NW


def _sc_gather_rows_body(tab_hbm, gi_hbm, out_hbm, idx_v, ebuf, sem):
    wid = lax.axis_index("s") * _NC + lax.axis_index("c")
    base = wid * _RPW
    pltpu.sync_copy(gi_hbm.at[pl.ds(base, _RPW)], idx_v)
    pltpu.async_copy(tab_hbm.at[idx_v], ebuf, sem).wait()
    pltpu.sync_copy(ebuf, out_hbm.at[pl.ds(base, _RPW)])


@functools.cache
def _sc_gather_rows_kernel():
    return pl.kernel(
        _sc_gather_rows_body,
        out_type=jax.ShapeDtypeStruct((B, D), jnp.float32),
        mesh=plsc.VectorSubcoreMesh(core_axis_name="c", subcore_axis_name="s",
                                    num_cores=_NC, num_subcores=_NS),
        scratch_types=[
            pltpu.VMEM((_RPW,), jnp.int32),
            pltpu.VMEM((_RPW, D), jnp.float32),
            pltpu.SemaphoreType.DMA,
        ],
    )


_BM_PROP = 400


def _prop_body(a_ref, ef_ref, er_ref, y_ref):
    y_ref[...] = er_ref[...] + jnp.dot(
        a_ref[...], ef_ref[...], preferred_element_type=jnp.float32)


def _propagate(adj, ebds):
    nblk = N // _BM_PROP
    return pl.pallas_call(
        _prop_body,
        grid=(nblk,),
        in_specs=[
            pl.BlockSpec((_BM_PROP, N), lambda i: (i, 0)),
            pl.BlockSpec((N, D), lambda i: (0, 0)),
            pl.BlockSpec((_BM_PROP, D), lambda i: (i, 0)),
        ],
        out_specs=pl.BlockSpec((_BM_PROP, D), lambda i: (i, 0)),
        out_shape=jax.ShapeDtypeStruct((N, D), jnp.float32),
        compiler_params=pltpu.CompilerParams(
            dimension_semantics=("arbitrary",)),
    )(adj, ebds, ebds)


_BM_MOB = 256


def _mob_body(gi_ref, adj_ref, y_ref, eg_ref, m_ref, mn_ref,
              buf0, buf1, sem0, sem1):
    i = pl.program_id(0)
    nblk = pl.num_programs(0)

    def dma(b, blk, buf, sem):
        row = gi_ref[blk * _BM_MOB + b]
        return pltpu.make_async_copy(
            adj_ref.at[pl.ds(row, 1)], buf.at[pl.ds(b, 1)], sem)

    def issue(blk, buf, sem):
        def f(b, c):
            dma(b, blk, buf, sem).start()
            return c
        lax.fori_loop(0, _BM_MOB, f, 0)

    def drain(blk, buf, sem):
        def f(b, c):
            dma(b, blk, buf, sem).wait()
            return c
        lax.fori_loop(0, _BM_MOB, f, 0)

    @pl.when(i == 0)
    def _():
        issue(0, buf0, sem0)

    even = i % 2 == 0

    @pl.when((i + 1 < nblk) & even)
    def _():
        issue(i + 1, buf1, sem1)

    @pl.when((i + 1 < nblk) & jnp.logical_not(even))
    def _():
        issue(i + 1, buf0, sem0)

    def compute(buf, sem):
        drain(i, buf, sem)
        m = eg_ref[...] + jnp.dot(
            buf[...], y_ref[...], preferred_element_type=jnp.float32)
        m_ref[...] = m
        mn_ref[...] = m / jnp.sqrt(jnp.sum(m * m, axis=1, keepdims=True))

    @pl.when(even)
    def _():
        compute(buf0, sem0)

    @pl.when(jnp.logical_not(even))
    def _():
        compute(buf1, sem1)


def _mob_embed(gi, adj, y, eg):
    nblk = B // _BM_MOB
    grid_spec = pltpu.PrefetchScalarGridSpec(
        num_scalar_prefetch=1,
        grid=(nblk,),
        in_specs=[
            pl.BlockSpec(memory_space=pl.ANY),
            pl.BlockSpec((N, D), lambda i, gi_ref: (0, 0)),
            pl.BlockSpec((_BM_MOB, D), lambda i, gi_ref: (i, 0)),
        ],
        out_specs=[
            pl.BlockSpec((_BM_MOB, D), lambda i, gi_ref: (i, 0)),
            pl.BlockSpec((_BM_MOB, D), lambda i, gi_ref: (i, 0)),
        ],
        scratch_shapes=[
            pltpu.VMEM((_BM_MOB, N), jnp.float32),
            pltpu.VMEM((_BM_MOB, N), jnp.float32),
            pltpu.SemaphoreType.DMA,
            pltpu.SemaphoreType.DMA,
        ],
    )
    return pl.pallas_call(
        _mob_body,
        grid_spec=grid_spec,
        out_shape=[
            jax.ShapeDtypeStruct((B, D), jnp.float32),
            jax.ShapeDtypeStruct((B, D), jnp.float32),
        ],
        compiler_params=pltpu.CompilerParams(
            dimension_semantics=("arbitrary",)),
    )(gi, adj, y, eg)


_BT = 512


def _towers_body(poi_ref, demo_ref, img_ref, wp_ref, bp_ref, w1_ref, b1_ref,
                 w2_ref, b2_ref, wi_ref, bi_ref, pn_ref, dn_ref, in_ref):
    def norm(x):
        return x / jnp.sqrt(jnp.sum(x * x, axis=1, keepdims=True))

    p = jnp.dot(poi_ref[...], wp_ref[...],
                preferred_element_type=jnp.float32) + bp_ref[...]
    pn_ref[...] = norm(p)
    h = jnp.maximum(jnp.dot(demo_ref[...], w1_ref[...],
                            preferred_element_type=jnp.float32) + b1_ref[...],
                    0.0)
    dd = jnp.dot(h, w2_ref[...],
                 preferred_element_type=jnp.float32) + b2_ref[...]
    dn_ref[...] = norm(dd)
    im = jnp.dot(img_ref[...], wi_ref[...],
                 preferred_element_type=jnp.float32) + bi_ref[...]
    in_ref[...] = norm(im)


def _towers(poi, demo, image, W_poi, b_poi, W1, b1, W2, b2, W_img, b_img):
    nblk = B // _BT
    poi_d, demo_d, img_d, demo_h = (W_poi.shape[0], W1.shape[0],
                                    W_img.shape[0], W1.shape[1])
    full = lambda shape: pl.BlockSpec(shape, lambda i: tuple(0 for _ in shape))
    return pl.pallas_call(
        _towers_body,
        grid=(nblk,),
        in_specs=[
            pl.BlockSpec((_BT, poi_d), lambda i: (i, 0)),
            pl.BlockSpec((_BT, demo_d), lambda i: (i, 0)),
            pl.BlockSpec((_BT, img_d), lambda i: (i, 0)),
            full((poi_d, D)), full((1, D)),
            full((demo_d, demo_h)), full((1, demo_h)),
            full((demo_h, D)), full((1, D)),
            full((img_d, D)), full((1, D)),
        ],
        out_specs=[pl.BlockSpec((_BT, D), lambda i: (i, 0))] * 3,
        out_shape=[jax.ShapeDtypeStruct((B, D), jnp.float32)] * 3,
        compiler_params=pltpu.CompilerParams(
            dimension_semantics=("arbitrary",)),
    )(poi, demo, image, W_poi, b_poi.reshape(1, -1), W1, b1.reshape(1, -1),
      W2, b2.reshape(1, -1), W_img, b_img.reshape(1, -1))


_BL = 512


def _logits_body(mn_ref, pn_ref, dn_ref, in_ref,
                 lmp_ref, lpm_ref, lmd_ref, ldm_ref, lmi_ref, lim_ref):
    def dg(a, b):
        return SCALE * lax.dot_general(
            a, b, (((1,), (1,)), ((), ())),
            preferred_element_type=jnp.float32)

    bi = pl.program_id(0) * _BL
    bj = pl.program_id(1) * _BL
    m = mn_ref[pl.ds(bi, _BL), :].astype(jnp.bfloat16)
    p = pn_ref[pl.ds(bj, _BL), :].astype(jnp.bfloat16)
    d = dn_ref[pl.ds(bj, _BL), :].astype(jnp.bfloat16)
    i = in_ref[pl.ds(bj, _BL), :].astype(jnp.bfloat16)
    lmp_ref[...] = dg(m, p)
    lpm_ref[...] = dg(p, m)
    lmd_ref[...] = dg(m, d)
    ldm_ref[...] = dg(d, m)
    lmi_ref[...] = dg(m, i)
    lim_ref[...] = dg(i, m)


def _logits(mn, pn, dn, imn):
    nblk = B // _BL
    full = pl.BlockSpec((B, D), lambda i, j: (0, 0))
    out_ij = pl.BlockSpec((_BL, _BL), lambda i, j: (i, j))
    out_ji = pl.BlockSpec((_BL, _BL), lambda i, j: (j, i))
    ls = jax.ShapeDtypeStruct((B, B), jnp.float32)
    return pl.pallas_call(
        _logits_body,
        grid=(nblk, nblk),
        in_specs=[full, full, full, full],
        out_specs=[out_ij, out_ji, out_ij, out_ji, out_ij, out_ji],
        out_shape=[ls] * 6,
        compiler_params=pltpu.CompilerParams(
            dimension_semantics=("parallel", "parallel")),
    )(mn, pn, dn, imn)


def kernel(poi, demo, image, mob_adj, global_indices, ebds,
           W_poi, b_poi, W1, b1, W2, b2, W_img, b_img):
    eg = _sc_gather_rows_kernel()(ebds, global_indices)
    y = _propagate(mob_adj, ebds)
    mob_ebd, mob_n = _mob_embed(global_indices, mob_adj, y, eg)
    poi_n, demo_n, img_n = _towers(poi, demo, image, W_poi, b_poi,
                                   W1, b1, W2, b2, W_img, b_img)
    l_mp, l_pm, l_md, l_dm, l_mi, l_im = _logits(mob_n, poi_n, demo_n, img_n)
    return (l_mp, l_pm, l_md, l_dm, l_mi, l_im, mob_ebd)


# speedup vs baseline: 1.3836x; 1.0102x over previous
import functools

import jax
import jax.numpy as jnp
from jax import lax
from jax.experimental import pallas as pl
from jax.experimental.pallas import tpu as pltpu
from jax.experimental.pallas import tpu_sc as plsc

N = 10000
D = 128
B = 4096
SCALE = 1.0 / 0.07

_NC, _NS = 2, 16
_NW = _NC * _NS
_RPW = B // _NW


def _sc_gather_rows_body(tab_hbm, gi_hbm, out_hbm, idx_v, ebuf, sem):
    wid = lax.axis_index("s") * _NC + lax.axis_index("c")
    base = wid * _RPW
    pltpu.sync_copy(gi_hbm.at[pl.ds(base, _RPW)], idx_v)
    pltpu.async_copy(tab_hbm.at[idx_v], ebuf, sem).wait()
    pltpu.sync_copy(ebuf, out_hbm.at[pl.ds(base, _RPW)])


@functools.cache
def _sc_gather_rows_kernel():
    return pl.kernel(
        _sc_gather_rows_body,
        out_type=jax.ShapeDtypeStruct((B, D), jnp.float32),
        mesh=plsc.VectorSubcoreMesh(core_axis_name="c", subcore_axis_name="s",
                                    num_cores=_NC, num_subcores=_NS),
        scratch_types=[
            pltpu.VMEM((_RPW,), jnp.int32),
            pltpu.VMEM((_RPW, D), jnp.float32),
            pltpu.SemaphoreType.DMA,
        ],
    )


_BM_PROP = 400


def _prop_body(a_ref, ef_ref, er_ref, y_ref):
    y_ref[...] = er_ref[...] + jnp.dot(
        a_ref[...], ef_ref[...], preferred_element_type=jnp.float32)


def _propagate(adj, ebds):
    nblk = N // _BM_PROP
    return pl.pallas_call(
        _prop_body,
        grid=(nblk,),
        in_specs=[
            pl.BlockSpec((_BM_PROP, N), lambda i: (i, 0)),
            pl.BlockSpec((N, D), lambda i: (0, 0)),
            pl.BlockSpec((_BM_PROP, D), lambda i: (i, 0)),
        ],
        out_specs=pl.BlockSpec((_BM_PROP, D), lambda i: (i, 0)),
        out_shape=jax.ShapeDtypeStruct((N, D), jnp.float32),
        compiler_params=pltpu.CompilerParams(
            dimension_semantics=("arbitrary",)),
    )(adj, ebds, ebds)


_BM_MOB = 512


def _mob_body(gi_ref, adj_ref, y_ref, eg_ref, m_ref, mn_ref,
              buf0, buf1, sem0, sem1):
    i = pl.program_id(0)
    nblk = pl.num_programs(0)

    def dma(b, blk, buf, sem):
        row = gi_ref[blk * _BM_MOB + b]
        return pltpu.make_async_copy(
            adj_ref.at[pl.ds(row, 1)], buf.at[pl.ds(b, 1)], sem)

    def issue(blk, buf, sem):
        def f(b, c):
            dma(b, blk, buf, sem).start()
            return c
        lax.fori_loop(0, _BM_MOB, f, 0)

    def drain(blk, buf, sem):
        def f(b, c):
            dma(b, blk, buf, sem).wait()
            return c
        lax.fori_loop(0, _BM_MOB, f, 0)

    @pl.when(i == 0)
    def _():
        issue(0, buf0, sem0)

    even = i % 2 == 0

    @pl.when((i + 1 < nblk) & even)
    def _():
        issue(i + 1, buf1, sem1)

    @pl.when((i + 1 < nblk) & jnp.logical_not(even))
    def _():
        issue(i + 1, buf0, sem0)

    def compute(buf, sem):
        drain(i, buf, sem)
        m = eg_ref[...] + jnp.dot(
            buf[...], y_ref[...], preferred_element_type=jnp.float32)
        m_ref[...] = m
        mn_ref[...] = m / jnp.sqrt(jnp.sum(m * m, axis=1, keepdims=True))

    @pl.when(even)
    def _():
        compute(buf0, sem0)

    @pl.when(jnp.logical_not(even))
    def _():
        compute(buf1, sem1)


def _mob_embed(gi, adj, y, eg):
    nblk = B // _BM_MOB
    grid_spec = pltpu.PrefetchScalarGridSpec(
        num_scalar_prefetch=1,
        grid=(nblk,),
        in_specs=[
            pl.BlockSpec(memory_space=pl.ANY),
            pl.BlockSpec((N, D), lambda i, gi_ref: (0, 0)),
            pl.BlockSpec((_BM_MOB, D), lambda i, gi_ref: (i, 0)),
        ],
        out_specs=[
            pl.BlockSpec((_BM_MOB, D), lambda i, gi_ref: (i, 0)),
            pl.BlockSpec((_BM_MOB, D), lambda i, gi_ref: (i, 0)),
        ],
        scratch_shapes=[
            pltpu.VMEM((_BM_MOB, N), jnp.float32),
            pltpu.VMEM((_BM_MOB, N), jnp.float32),
            pltpu.SemaphoreType.DMA,
            pltpu.SemaphoreType.DMA,
        ],
    )
    return pl.pallas_call(
        _mob_body,
        grid_spec=grid_spec,
        out_shape=[
            jax.ShapeDtypeStruct((B, D), jnp.float32),
            jax.ShapeDtypeStruct((B, D), jnp.float32),
        ],
        compiler_params=pltpu.CompilerParams(
            dimension_semantics=("arbitrary",)),
    )(gi, adj, y, eg)


_BT = 512


def _towers_body(poi_ref, demo_ref, img_ref, wp_ref, bp_ref, w1_ref, b1_ref,
                 w2_ref, b2_ref, wi_ref, bi_ref, pn_ref, dn_ref, in_ref):
    def norm(x):
        return x / jnp.sqrt(jnp.sum(x * x, axis=1, keepdims=True))

    p = jnp.dot(poi_ref[...], wp_ref[...],
                preferred_element_type=jnp.float32) + bp_ref[...]
    pn_ref[...] = norm(p)
    h = jnp.maximum(jnp.dot(demo_ref[...], w1_ref[...],
                            preferred_element_type=jnp.float32) + b1_ref[...],
                    0.0)
    dd = jnp.dot(h, w2_ref[...],
                 preferred_element_type=jnp.float32) + b2_ref[...]
    dn_ref[...] = norm(dd)
    im = jnp.dot(img_ref[...], wi_ref[...],
                 preferred_element_type=jnp.float32) + bi_ref[...]
    in_ref[...] = norm(im)


def _towers(poi, demo, image, W_poi, b_poi, W1, b1, W2, b2, W_img, b_img):
    nblk = B // _BT
    poi_d, demo_d, img_d, demo_h = (W_poi.shape[0], W1.shape[0],
                                    W_img.shape[0], W1.shape[1])
    full = lambda shape: pl.BlockSpec(shape, lambda i: tuple(0 for _ in shape))
    return pl.pallas_call(
        _towers_body,
        grid=(nblk,),
        in_specs=[
            pl.BlockSpec((_BT, poi_d), lambda i: (i, 0)),
            pl.BlockSpec((_BT, demo_d), lambda i: (i, 0)),
            pl.BlockSpec((_BT, img_d), lambda i: (i, 0)),
            full((poi_d, D)), full((1, D)),
            full((demo_d, demo_h)), full((1, demo_h)),
            full((demo_h, D)), full((1, D)),
            full((img_d, D)), full((1, D)),
        ],
        out_specs=[pl.BlockSpec((_BT, D), lambda i: (i, 0))] * 3,
        out_shape=[jax.ShapeDtypeStruct((B, D), jnp.float32)] * 3,
        compiler_params=pltpu.CompilerParams(
            dimension_semantics=("arbitrary",)),
    )(poi, demo, image, W_poi, b_poi.reshape(1, -1), W1, b1.reshape(1, -1),
      W2, b2.reshape(1, -1), W_img, b_img.reshape(1, -1))


_BL = 512


def _logits_body(mn_ref, pn_ref, dn_ref, in_ref,
                 lmp_ref, lpm_ref, lmd_ref, ldm_ref, lmi_ref, lim_ref):
    def dg(a, b):
        return SCALE * lax.dot_general(
            a, b, (((1,), (1,)), ((), ())),
            preferred_element_type=jnp.float32)

    bi = pl.program_id(0) * _BL
    bj = pl.program_id(1) * _BL
    m = mn_ref[pl.ds(bi, _BL), :].astype(jnp.bfloat16)
    p = pn_ref[pl.ds(bj, _BL), :].astype(jnp.bfloat16)
    d = dn_ref[pl.ds(bj, _BL), :].astype(jnp.bfloat16)
    i = in_ref[pl.ds(bj, _BL), :].astype(jnp.bfloat16)
    lmp_ref[...] = dg(m, p)
    lpm_ref[...] = dg(p, m)
    lmd_ref[...] = dg(m, d)
    ldm_ref[...] = dg(d, m)
    lmi_ref[...] = dg(m, i)
    lim_ref[...] = dg(i, m)


def _logits(mn, pn, dn, imn):
    nblk = B // _BL
    full = pl.BlockSpec((B, D), lambda i, j: (0, 0))
    out_ij = pl.BlockSpec((_BL, _BL), lambda i, j: (i, j))
    out_ji = pl.BlockSpec((_BL, _BL), lambda i, j: (j, i))
    ls = jax.ShapeDtypeStruct((B, B), jnp.float32)
    return pl.pallas_call(
        _logits_body,
        grid=(nblk, nblk),
        in_specs=[full, full, full, full],
        out_specs=[out_ij, out_ji, out_ij, out_ji, out_ij, out_ji],
        out_shape=[ls] * 6,
        compiler_params=pltpu.CompilerParams(
            dimension_semantics=("parallel", "parallel")),
    )(mn, pn, dn, imn)


def kernel(poi, demo, image, mob_adj, global_indices, ebds,
           W_poi, b_poi, W1, b1, W2, b2, W_img, b_img):
    eg = _sc_gather_rows_kernel()(ebds, global_indices)
    y = _propagate(mob_adj, ebds)
    mob_ebd, mob_n = _mob_embed(global_indices, mob_adj, y, eg)
    poi_n, demo_n, img_n = _towers(poi, demo, image, W_poi, b_poi,
                                   W1, b1, W2, b2, W_img, b_img)
    l_mp, l_pm, l_md, l_dm, l_mi, l_im = _logits(mob_n, poi_n, demo_n, img_n)
    return (l_mp, l_pm, l_md, l_dm, l_mi, l_im, mob_ebd)


# speedup vs baseline: 1.4992x; 1.0835x over previous
import functools

import jax
import jax.numpy as jnp
from jax import lax
from jax.experimental import pallas as pl
from jax.experimental.pallas import tpu as pltpu
from jax.experimental.pallas import tpu_sc as plsc

N = 10000
D = 128
B = 4096
SCALE = 1.0 / 0.07

_NC, _NS = 2, 16
_NW = _NC * _NS
_RPW = B // _NW


def _sc_gather_rows_body(tab_hbm, gi_hbm, out_hbm, idx_v, ebuf, sem):
    wid = lax.axis_index("s") * _NC + lax.axis_index("c")
    base = wid * _RPW
    pltpu.sync_copy(gi_hbm.at[pl.ds(base, _RPW)], idx_v)
    pltpu.async_copy(tab_hbm.at[idx_v], ebuf, sem).wait()
    pltpu.sync_copy(ebuf, out_hbm.at[pl.ds(base, _RPW)])


@functools.cache
def _sc_gather_rows_kernel():
    return pl.kernel(
        _sc_gather_rows_body,
        out_type=jax.ShapeDtypeStruct((B, D), jnp.float32),
        mesh=plsc.VectorSubcoreMesh(core_axis_name="c", subcore_axis_name="s",
                                    num_cores=_NC, num_subcores=_NS),
        scratch_types=[
            pltpu.VMEM((_RPW,), jnp.int32),
            pltpu.VMEM((_RPW, D), jnp.float32),
            pltpu.SemaphoreType.DMA,
        ],
    )


_BM_PROP = 400


def _prop_body(a_ref, ef_ref, er_ref, y_ref):
    y_ref[...] = er_ref[...] + jnp.dot(
        a_ref[...], ef_ref[...], preferred_element_type=jnp.float32)


def _propagate(adj, ebds):
    nblk = N // _BM_PROP
    return pl.pallas_call(
        _prop_body,
        grid=(nblk,),
        in_specs=[
            pl.BlockSpec((_BM_PROP, N), lambda i: (i, 0)),
            pl.BlockSpec((N, D), lambda i: (0, 0)),
            pl.BlockSpec((_BM_PROP, D), lambda i: (i, 0)),
        ],
        out_specs=pl.BlockSpec((_BM_PROP, D), lambda i: (i, 0)),
        out_shape=jax.ShapeDtypeStruct((N, D), jnp.float32),
        compiler_params=pltpu.CompilerParams(
            dimension_semantics=("arbitrary",)),
    )(adj, ebds, ebds)


_BM_MOB = 512


def _mob_body(gi_ref, adj_ref, y_ref, eg_ref, m_ref, mn_ref,
              buf0, buf1, sem0, sem1):
    i = pl.program_id(0)
    nblk = pl.num_programs(0)

    def dma(b, k, blk, buf, sem):
        row = gi_ref[blk * _BM_MOB + b]
        return pltpu.make_async_copy(
            adj_ref.at[pl.ds(row, 1)], buf.at[pl.ds(b, 1)], sem.at[k])

    def issue(blk, buf, sem):
        def f(g, c):
            for k in range(8):
                dma(g * 8 + k, k, blk, buf, sem).start()
            return c
        lax.fori_loop(0, _BM_MOB // 8, f, 0)

    def drain(blk, buf, sem):
        def f(g, c):
            for k in range(8):
                dma(g * 8 + k, k, blk, buf, sem).wait()
            return c
        lax.fori_loop(0, _BM_MOB // 8, f, 0)

    @pl.when(i == 0)
    def _():
        issue(0, buf0, sem0)

    even = i % 2 == 0

    @pl.when((i + 1 < nblk) & even)
    def _():
        issue(i + 1, buf1, sem1)

    @pl.when((i + 1 < nblk) & jnp.logical_not(even))
    def _():
        issue(i + 1, buf0, sem0)

    def compute(buf, sem):
        drain(i, buf, sem)
        m = eg_ref[...] + jnp.dot(
            buf[...], y_ref[...], preferred_element_type=jnp.float32)
        m_ref[...] = m
        mn_ref[...] = m / jnp.sqrt(jnp.sum(m * m, axis=1, keepdims=True))

    @pl.when(even)
    def _():
        compute(buf0, sem0)

    @pl.when(jnp.logical_not(even))
    def _():
        compute(buf1, sem1)


def _mob_embed(gi, adj, y, eg):
    nblk = B // _BM_MOB
    grid_spec = pltpu.PrefetchScalarGridSpec(
        num_scalar_prefetch=1,
        grid=(nblk,),
        in_specs=[
            pl.BlockSpec(memory_space=pl.ANY),
            pl.BlockSpec((N, D), lambda i, gi_ref: (0, 0)),
            pl.BlockSpec((_BM_MOB, D), lambda i, gi_ref: (i, 0)),
        ],
        out_specs=[
            pl.BlockSpec((_BM_MOB, D), lambda i, gi_ref: (i, 0)),
            pl.BlockSpec((_BM_MOB, D), lambda i, gi_ref: (i, 0)),
        ],
        scratch_shapes=[
            pltpu.VMEM((_BM_MOB, N), jnp.float32),
            pltpu.VMEM((_BM_MOB, N), jnp.float32),
            pltpu.SemaphoreType.DMA((8,)),
            pltpu.SemaphoreType.DMA((8,)),
        ],
    )
    return pl.pallas_call(
        _mob_body,
        grid_spec=grid_spec,
        out_shape=[
            jax.ShapeDtypeStruct((B, D), jnp.float32),
            jax.ShapeDtypeStruct((B, D), jnp.float32),
        ],
        compiler_params=pltpu.CompilerParams(
            dimension_semantics=("arbitrary",)),
    )(gi, adj, y, eg)


_BT = 512


def _towers_body(poi_ref, demo_ref, img_ref, wp_ref, bp_ref, w1_ref, b1_ref,
                 w2_ref, b2_ref, wi_ref, bi_ref, pn_ref, dn_ref, in_ref):
    def norm(x):
        return x / jnp.sqrt(jnp.sum(x * x, axis=1, keepdims=True))

    p = jnp.dot(poi_ref[...], wp_ref[...],
                preferred_element_type=jnp.float32) + bp_ref[...]
    pn_ref[...] = norm(p)
    h = jnp.maximum(jnp.dot(demo_ref[...], w1_ref[...],
                            preferred_element_type=jnp.float32) + b1_ref[...],
                    0.0)
    dd = jnp.dot(h, w2_ref[...],
                 preferred_element_type=jnp.float32) + b2_ref[...]
    dn_ref[...] = norm(dd)
    im = jnp.dot(img_ref[...], wi_ref[...],
                 preferred_element_type=jnp.float32) + bi_ref[...]
    in_ref[...] = norm(im)


def _towers(poi, demo, image, W_poi, b_poi, W1, b1, W2, b2, W_img, b_img):
    nblk = B // _BT
    poi_d, demo_d, img_d, demo_h = (W_poi.shape[0], W1.shape[0],
                                    W_img.shape[0], W1.shape[1])
    full = lambda shape: pl.BlockSpec(shape, lambda i: tuple(0 for _ in shape))
    return pl.pallas_call(
        _towers_body,
        grid=(nblk,),
        in_specs=[
            pl.BlockSpec((_BT, poi_d), lambda i: (i, 0)),
            pl.BlockSpec((_BT, demo_d), lambda i: (i, 0)),
            pl.BlockSpec((_BT, img_d), lambda i: (i, 0)),
            full((poi_d, D)), full((1, D)),
            full((demo_d, demo_h)), full((1, demo_h)),
            full((demo_h, D)), full((1, D)),
            full((img_d, D)), full((1, D)),
        ],
        out_specs=[pl.BlockSpec((_BT, D), lambda i: (i, 0))] * 3,
        out_shape=[jax.ShapeDtypeStruct((B, D), jnp.float32)] * 3,
        compiler_params=pltpu.CompilerParams(
            dimension_semantics=("arbitrary",)),
    )(poi, demo, image, W_poi, b_poi.reshape(1, -1), W1, b1.reshape(1, -1),
      W2, b2.reshape(1, -1), W_img, b_img.reshape(1, -1))


_BL = 512


def _logits_body(mn_ref, pn_ref, dn_ref, in_ref,
                 lmp_ref, lpm_ref, lmd_ref, ldm_ref, lmi_ref, lim_ref):
    def dg(a, b):
        return SCALE * lax.dot_general(
            a, b, (((1,), (1,)), ((), ())),
            preferred_element_type=jnp.float32)

    bi = pl.program_id(0) * _BL
    bj = pl.program_id(1) * _BL
    m = mn_ref[pl.ds(bi, _BL), :].astype(jnp.bfloat16)
    p = pn_ref[pl.ds(bj, _BL), :].astype(jnp.bfloat16)
    d = dn_ref[pl.ds(bj, _BL), :].astype(jnp.bfloat16)
    i = in_ref[pl.ds(bj, _BL), :].astype(jnp.bfloat16)
    lmp_ref[...] = dg(m, p)
    lpm_ref[...] = dg(p, m)
    lmd_ref[...] = dg(m, d)
    ldm_ref[...] = dg(d, m)
    lmi_ref[...] = dg(m, i)
    lim_ref[...] = dg(i, m)


def _logits(mn, pn, dn, imn):
    nblk = B // _BL
    full = pl.BlockSpec((B, D), lambda i, j: (0, 0))
    out_ij = pl.BlockSpec((_BL, _BL), lambda i, j: (i, j))
    out_ji = pl.BlockSpec((_BL, _BL), lambda i, j: (j, i))
    ls = jax.ShapeDtypeStruct((B, B), jnp.float32)
    return pl.pallas_call(
        _logits_body,
        grid=(nblk, nblk),
        in_specs=[full, full, full, full],
        out_specs=[out_ij, out_ji, out_ij, out_ji, out_ij, out_ji],
        out_shape=[ls] * 6,
        compiler_params=pltpu.CompilerParams(
            dimension_semantics=("parallel", "parallel")),
    )(mn, pn, dn, imn)


def kernel(poi, demo, image, mob_adj, global_indices, ebds,
           W_poi, b_poi, W1, b1, W2, b2, W_img, b_img):
    eg = _sc_gather_rows_kernel()(ebds, global_indices)
    y = _propagate(mob_adj, ebds)
    mob_ebd, mob_n = _mob_embed(global_indices, mob_adj, y, eg)
    poi_n, demo_n, img_n = _towers(poi, demo, image, W_poi, b_poi,
                                   W1, b1, W2, b2, W_img, b_img)
    l_mp, l_pm, l_md, l_dm, l_mi, l_im = _logits(mob_n, poi_n, demo_n, img_n)
    return (l_mp, l_pm, l_md, l_dm, l_mi, l_im, mob_ebd)


# speedup vs baseline: 1.5035x; 1.0028x over previous
import functools

import jax
import jax.numpy as jnp
from jax import lax
from jax.experimental import pallas as pl
from jax.experimental.pallas import tpu as pltpu
from jax.experimental.pallas import tpu_sc as plsc

N = 10000
D = 128
B = 4096
SCALE = 1.0 / 0.07

_NC, _NS = 2, 16
_NW = _NC * _NS
_RPW = B // _NW


def _sc_gather_rows_body(tab_hbm, gi_hbm, out_hbm, idx_v, ebuf, sem):
    wid = lax.axis_index("s") * _NC + lax.axis_index("c")
    base = wid * _RPW
    pltpu.sync_copy(gi_hbm.at[pl.ds(base, _RPW)], idx_v)
    pltpu.async_copy(tab_hbm.at[idx_v], ebuf, sem).wait()
    pltpu.sync_copy(ebuf, out_hbm.at[pl.ds(base, _RPW)])


@functools.cache
def _sc_gather_rows_kernel():
    return pl.kernel(
        _sc_gather_rows_body,
        out_type=jax.ShapeDtypeStruct((B, D), jnp.float32),
        mesh=plsc.VectorSubcoreMesh(core_axis_name="c", subcore_axis_name="s",
                                    num_cores=_NC, num_subcores=_NS),
        scratch_types=[
            pltpu.VMEM((_RPW,), jnp.int32),
            pltpu.VMEM((_RPW, D), jnp.float32),
            pltpu.SemaphoreType.DMA,
        ],
    )


_BM_PROP = 400


def _prop_body(a_ref, ef_ref, er_ref, y_ref):
    y_ref[...] = er_ref[...] + jnp.dot(
        a_ref[...], ef_ref[...], preferred_element_type=jnp.float32)


def _propagate(adj, ebds):
    nblk = N // _BM_PROP
    return pl.pallas_call(
        _prop_body,
        grid=(nblk,),
        in_specs=[
            pl.BlockSpec((_BM_PROP, N), lambda i: (i, 0)),
            pl.BlockSpec((N, D), lambda i: (0, 0)),
            pl.BlockSpec((_BM_PROP, D), lambda i: (i, 0)),
        ],
        out_specs=pl.BlockSpec((_BM_PROP, D), lambda i: (i, 0)),
        out_shape=jax.ShapeDtypeStruct((N, D), jnp.float32),
        compiler_params=pltpu.CompilerParams(
            dimension_semantics=("arbitrary",)),
    )(adj, ebds, ebds)


_BM_MOB = 512


def _mob_body(gi_ref, adj_ref, y_ref, eg_ref, m_ref, mn_ref,
              buf0, buf1, sem0, sem1):
    i = pl.program_id(0)
    nblk = pl.num_programs(0)

    def dma(b, k, blk, buf, sem):
        row = gi_ref[blk * _BM_MOB + b]
        return pltpu.make_async_copy(
            adj_ref.at[pl.ds(row, 1)], buf.at[pl.ds(b, 1)], sem.at[k])

    def issue(blk, buf, sem):
        def f(g, c):
            for k in range(16):
                dma(g * 16 + k, k, blk, buf, sem).start()
            return c
        lax.fori_loop(0, _BM_MOB // 16, f, 0)

    def drain(blk, buf, sem):
        def f(g, c):
            for k in range(16):
                dma(g * 16 + k, k, blk, buf, sem).wait()
            return c
        lax.fori_loop(0, _BM_MOB // 16, f, 0)

    @pl.when(i == 0)
    def _():
        issue(0, buf0, sem0)

    even = i % 2 == 0

    @pl.when((i + 1 < nblk) & even)
    def _():
        issue(i + 1, buf1, sem1)

    @pl.when((i + 1 < nblk) & jnp.logical_not(even))
    def _():
        issue(i + 1, buf0, sem0)

    def compute(buf, sem):
        drain(i, buf, sem)
        m = eg_ref[...] + jnp.dot(
            buf[...], y_ref[...], preferred_element_type=jnp.float32)
        m_ref[...] = m
        mn_ref[...] = m / jnp.sqrt(jnp.sum(m * m, axis=1, keepdims=True))

    @pl.when(even)
    def _():
        compute(buf0, sem0)

    @pl.when(jnp.logical_not(even))
    def _():
        compute(buf1, sem1)


def _mob_embed(gi, adj, y, eg):
    nblk = B // _BM_MOB
    grid_spec = pltpu.PrefetchScalarGridSpec(
        num_scalar_prefetch=1,
        grid=(nblk,),
        in_specs=[
            pl.BlockSpec(memory_space=pl.ANY),
            pl.BlockSpec((N, D), lambda i, gi_ref: (0, 0)),
            pl.BlockSpec((_BM_MOB, D), lambda i, gi_ref: (i, 0)),
        ],
        out_specs=[
            pl.BlockSpec((_BM_MOB, D), lambda i, gi_ref: (i, 0)),
            pl.BlockSpec((_BM_MOB, D), lambda i, gi_ref: (i, 0)),
        ],
        scratch_shapes=[
            pltpu.VMEM((_BM_MOB, N), jnp.float32),
            pltpu.VMEM((_BM_MOB, N), jnp.float32),
            pltpu.SemaphoreType.DMA((16,)),
            pltpu.SemaphoreType.DMA((16,)),
        ],
    )
    return pl.pallas_call(
        _mob_body,
        grid_spec=grid_spec,
        out_shape=[
            jax.ShapeDtypeStruct((B, D), jnp.float32),
            jax.ShapeDtypeStruct((B, D), jnp.float32),
        ],
        compiler_params=pltpu.CompilerParams(
            dimension_semantics=("arbitrary",)),
    )(gi, adj, y, eg)


_BT = 512


def _towers_body(poi_ref, demo_ref, img_ref, wp_ref, bp_ref, w1_ref, b1_ref,
                 w2_ref, b2_ref, wi_ref, bi_ref, pn_ref, dn_ref, in_ref):
    def norm(x):
        return x / jnp.sqrt(jnp.sum(x * x, axis=1, keepdims=True))

    p = jnp.dot(poi_ref[...], wp_ref[...],
                preferred_element_type=jnp.float32) + bp_ref[...]
    pn_ref[...] = norm(p)
    h = jnp.maximum(jnp.dot(demo_ref[...], w1_ref[...],
                            preferred_element_type=jnp.float32) + b1_ref[...],
                    0.0)
    dd = jnp.dot(h, w2_ref[...],
                 preferred_element_type=jnp.float32) + b2_ref[...]
    dn_ref[...] = norm(dd)
    im = jnp.dot(img_ref[...], wi_ref[...],
                 preferred_element_type=jnp.float32) + bi_ref[...]
    in_ref[...] = norm(im)


def _towers(poi, demo, image, W_poi, b_poi, W1, b1, W2, b2, W_img, b_img):
    nblk = B // _BT
    poi_d, demo_d, img_d, demo_h = (W_poi.shape[0], W1.shape[0],
                                    W_img.shape[0], W1.shape[1])
    full = lambda shape: pl.BlockSpec(shape, lambda i: tuple(0 for _ in shape))
    return pl.pallas_call(
        _towers_body,
        grid=(nblk,),
        in_specs=[
            pl.BlockSpec((_BT, poi_d), lambda i: (i, 0)),
            pl.BlockSpec((_BT, demo_d), lambda i: (i, 0)),
            pl.BlockSpec((_BT, img_d), lambda i: (i, 0)),
            full((poi_d, D)), full((1, D)),
            full((demo_d, demo_h)), full((1, demo_h)),
            full((demo_h, D)), full((1, D)),
            full((img_d, D)), full((1, D)),
        ],
        out_specs=[pl.BlockSpec((_BT, D), lambda i: (i, 0))] * 3,
        out_shape=[jax.ShapeDtypeStruct((B, D), jnp.float32)] * 3,
        compiler_params=pltpu.CompilerParams(
            dimension_semantics=("arbitrary",)),
    )(poi, demo, image, W_poi, b_poi.reshape(1, -1), W1, b1.reshape(1, -1),
      W2, b2.reshape(1, -1), W_img, b_img.reshape(1, -1))


_BL = 512


def _logits_body(mn_ref, pn_ref, dn_ref, in_ref,
                 lmp_ref, lpm_ref, lmd_ref, ldm_ref, lmi_ref, lim_ref):
    def dg(a, b):
        return SCALE * lax.dot_general(
            a, b, (((1,), (1,)), ((), ())),
            preferred_element_type=jnp.float32)

    bi = pl.program_id(0) * _BL
    bj = pl.program_id(1) * _BL
    m = mn_ref[pl.ds(bi, _BL), :].astype(jnp.bfloat16)
    p = pn_ref[pl.ds(bj, _BL), :].astype(jnp.bfloat16)
    d = dn_ref[pl.ds(bj, _BL), :].astype(jnp.bfloat16)
    i = in_ref[pl.ds(bj, _BL), :].astype(jnp.bfloat16)
    lmp_ref[...] = dg(m, p)
    lpm_ref[...] = dg(p, m)
    lmd_ref[...] = dg(m, d)
    ldm_ref[...] = dg(d, m)
    lmi_ref[...] = dg(m, i)
    lim_ref[...] = dg(i, m)


def _logits(mn, pn, dn, imn):
    nblk = B // _BL
    full = pl.BlockSpec((B, D), lambda i, j: (0, 0))
    out_ij = pl.BlockSpec((_BL, _BL), lambda i, j: (i, j))
    out_ji = pl.BlockSpec((_BL, _BL), lambda i, j: (j, i))
    ls = jax.ShapeDtypeStruct((B, B), jnp.float32)
    return pl.pallas_call(
        _logits_body,
        grid=(nblk, nblk),
        in_specs=[full, full, full, full],
        out_specs=[out_ij, out_ji, out_ij, out_ji, out_ij, out_ji],
        out_shape=[ls] * 6,
        compiler_params=pltpu.CompilerParams(
            dimension_semantics=("parallel", "parallel")),
    )(mn, pn, dn, imn)


def kernel(poi, demo, image, mob_adj, global_indices, ebds,
           W_poi, b_poi, W1, b1, W2, b2, W_img, b_img):
    eg = _sc_gather_rows_kernel()(ebds, global_indices)
    y = _propagate(mob_adj, ebds)
    mob_ebd, mob_n = _mob_embed(global_indices, mob_adj, y, eg)
    poi_n, demo_n, img_n = _towers(poi, demo, image, W_poi, b_poi,
                                   W1, b1, W2, b2, W_img, b_img)
    l_mp, l_pm, l_md, l_dm, l_mi, l_im = _logits(mob_n, poi_n, demo_n, img_n)
    return (l_mp, l_pm, l_md, l_dm, l_mi, l_im, mob_ebd)


# speedup vs baseline: 1.5192x; 1.0105x over previous
import functools

import jax
import jax.numpy as jnp
from jax import lax
from jax.experimental import pallas as pl
from jax.experimental.pallas import tpu as pltpu
from jax.experimental.pallas import tpu_sc as plsc

N = 10000
D = 128
B = 4096
SCALE = 1.0 / 0.07

_NC, _NS = 2, 16
_NW = _NC * _NS
_RPW = B // _NW


def _sc_gather_rows_body(tab_hbm, gi_hbm, out_hbm, idx_v, ebuf, sem):
    wid = lax.axis_index("s") * _NC + lax.axis_index("c")
    base = wid * _RPW
    pltpu.sync_copy(gi_hbm.at[pl.ds(base, _RPW)], idx_v)
    pltpu.async_copy(tab_hbm.at[idx_v], ebuf, sem).wait()
    pltpu.sync_copy(ebuf, out_hbm.at[pl.ds(base, _RPW)])


@functools.cache
def _sc_gather_rows_kernel():
    return pl.kernel(
        _sc_gather_rows_body,
        out_type=jax.ShapeDtypeStruct((B, D), jnp.float32),
        mesh=plsc.VectorSubcoreMesh(core_axis_name="c", subcore_axis_name="s",
                                    num_cores=_NC, num_subcores=_NS),
        scratch_types=[
            pltpu.VMEM((_RPW,), jnp.int32),
            pltpu.VMEM((_RPW, D), jnp.float32),
            pltpu.SemaphoreType.DMA,
        ],
    )


_BM_PROP = 400


def _prop_body(a_ref, ef_ref, er_ref, y_ref):
    y_ref[...] = er_ref[...] + jnp.dot(
        a_ref[...], ef_ref[...], preferred_element_type=jnp.float32)


def _propagate(adj, ebds):
    nblk = N // _BM_PROP
    return pl.pallas_call(
        _prop_body,
        grid=(nblk,),
        in_specs=[
            pl.BlockSpec((_BM_PROP, N), lambda i: (i, 0)),
            pl.BlockSpec((N, D), lambda i: (0, 0)),
            pl.BlockSpec((_BM_PROP, D), lambda i: (i, 0)),
        ],
        out_specs=pl.BlockSpec((_BM_PROP, D), lambda i: (i, 0)),
        out_shape=jax.ShapeDtypeStruct((N, D), jnp.float32),
        compiler_params=pltpu.CompilerParams(
            dimension_semantics=("arbitrary",)),
    )(adj, ebds, ebds)


_BM_MOB = 512


def _mob_body(gi_ref, adj_ref, y_ref, eg_ref, m_ref, mn_ref,
              buf0, buf1, sem0, sem1):
    i = pl.program_id(0)
    nblk = pl.num_programs(0)

    def dma(b, k, blk, buf, sem):
        row = gi_ref[blk * _BM_MOB + b]
        return pltpu.make_async_copy(
            adj_ref.at[pl.ds(row, 1)], buf.at[pl.ds(b, 1)], sem.at[k])

    def issue(blk, buf, sem):
        def f(g, c):
            for k in range(16):
                dma(g * 16 + k, k, blk, buf, sem).start()
            return c
        lax.fori_loop(0, _BM_MOB // 16, f, 0)

    def drain(blk, buf, sem):
        def f(g, c):
            for k in range(16):
                dma(g * 16 + k, k, blk, buf, sem).wait()
            return c
        lax.fori_loop(0, _BM_MOB // 16, f, 0)

    @pl.when(i == 0)
    def _():
        issue(0, buf0, sem0)

    even = i % 2 == 0

    @pl.when((i + 1 < nblk) & even)
    def _():
        issue(i + 1, buf1, sem1)

    @pl.when((i + 1 < nblk) & jnp.logical_not(even))
    def _():
        issue(i + 1, buf0, sem0)

    def compute(buf, sem):
        drain(i, buf, sem)
        m = eg_ref[...] + jnp.dot(
            buf[...], y_ref[...], preferred_element_type=jnp.float32)
        m_ref[...] = m
        mn_ref[...] = m / jnp.sqrt(jnp.sum(m * m, axis=1, keepdims=True))

    @pl.when(even)
    def _():
        compute(buf0, sem0)

    @pl.when(jnp.logical_not(even))
    def _():
        compute(buf1, sem1)


def _mob_embed(gi, adj, y, eg):
    nblk = B // _BM_MOB
    grid_spec = pltpu.PrefetchScalarGridSpec(
        num_scalar_prefetch=1,
        grid=(nblk,),
        in_specs=[
            pl.BlockSpec(memory_space=pl.ANY),
            pl.BlockSpec((N, D), lambda i, gi_ref: (0, 0)),
            pl.BlockSpec((_BM_MOB, D), lambda i, gi_ref: (i, 0)),
        ],
        out_specs=[
            pl.BlockSpec((_BM_MOB, D), lambda i, gi_ref: (i, 0)),
            pl.BlockSpec((_BM_MOB, D), lambda i, gi_ref: (i, 0)),
        ],
        scratch_shapes=[
            pltpu.VMEM((_BM_MOB, N), jnp.float32),
            pltpu.VMEM((_BM_MOB, N), jnp.float32),
            pltpu.SemaphoreType.DMA((16,)),
            pltpu.SemaphoreType.DMA((16,)),
        ],
    )
    return pl.pallas_call(
        _mob_body,
        grid_spec=grid_spec,
        out_shape=[
            jax.ShapeDtypeStruct((B, D), jnp.float32),
            jax.ShapeDtypeStruct((B, D), jnp.float32),
        ],
        compiler_params=pltpu.CompilerParams(
            dimension_semantics=("arbitrary",)),
    )(gi, adj, y, eg)


_BL = 512


def _logits_body(mn_ref, poi_ref, demo_ref, img_ref, wp_ref, bp_ref, w1_ref,
                 b1_ref, w2_ref, b2_ref, wi_ref, bi_ref,
                 lmp_ref, lpm_ref, lmd_ref, ldm_ref, lmi_ref, lim_ref,
                 pn_s, dn_s, in_s):
    bi = pl.program_id(0) * _BL
    bj = pl.program_id(1) * _BL

    @pl.when(pl.program_id(0) == 0)
    def _():
        def norm16(x):
            return (x / jnp.sqrt(jnp.sum(x * x, axis=1, keepdims=True))
                    ).astype(jnp.bfloat16)

        p = jnp.dot(poi_ref[pl.ds(bj, _BL), :], wp_ref[...],
                    preferred_element_type=jnp.float32) + bp_ref[...]
        pn_s[pl.ds(bj, _BL), :] = norm16(p)
        h = jnp.maximum(
            jnp.dot(demo_ref[pl.ds(bj, _BL), :], w1_ref[...],
                    preferred_element_type=jnp.float32) + b1_ref[...], 0.0)
        dd = jnp.dot(h, w2_ref[...],
                     preferred_element_type=jnp.float32) + b2_ref[...]
        dn_s[pl.ds(bj, _BL), :] = norm16(dd)
        im = jnp.dot(img_ref[pl.ds(bj, _BL), :], wi_ref[...],
                     preferred_element_type=jnp.float32) + bi_ref[...]
        in_s[pl.ds(bj, _BL), :] = norm16(im)

    def dg(a, b):
        return SCALE * lax.dot_general(
            a, b, (((1,), (1,)), ((), ())),
            preferred_element_type=jnp.float32)

    m = mn_ref[pl.ds(bi, _BL), :].astype(jnp.bfloat16)
    p = pn_s[pl.ds(bj, _BL), :]
    d = dn_s[pl.ds(bj, _BL), :]
    i = in_s[pl.ds(bj, _BL), :]
    lmp_ref[...] = dg(m, p)
    lpm_ref[...] = dg(p, m)
    lmd_ref[...] = dg(m, d)
    ldm_ref[...] = dg(d, m)
    lmi_ref[...] = dg(m, i)
    lim_ref[...] = dg(i, m)


def _logits(mn, poi, demo, image, W_poi, b_poi, W1, b1, W2, b2, W_img, b_img):
    nblk = B // _BL
    poi_d, demo_d, img_d, demo_h = (W_poi.shape[0], W1.shape[0],
                                    W_img.shape[0], W1.shape[1])
    full = lambda shape: pl.BlockSpec(
        shape, lambda i, j: tuple(0 for _ in shape))
    out_ij = pl.BlockSpec((_BL, _BL), lambda i, j: (i, j))
    out_ji = pl.BlockSpec((_BL, _BL), lambda i, j: (j, i))
    ls = jax.ShapeDtypeStruct((B, B), jnp.float32)
    return pl.pallas_call(
        _logits_body,
        grid=(nblk, nblk),
        in_specs=[
            full((B, D)), full((B, poi_d)), full((B, demo_d)),
            full((B, img_d)),
            full((poi_d, D)), full((1, D)),
            full((demo_d, demo_h)), full((1, demo_h)),
            full((demo_h, D)), full((1, D)),
            full((img_d, D)), full((1, D)),
        ],
        out_specs=[out_ij, out_ji, out_ij, out_ji, out_ij, out_ji],
        out_shape=[ls] * 6,
        scratch_shapes=[
            pltpu.VMEM((B, D), jnp.bfloat16),
            pltpu.VMEM((B, D), jnp.bfloat16),
            pltpu.VMEM((B, D), jnp.bfloat16),
        ],
        compiler_params=pltpu.CompilerParams(
            dimension_semantics=("arbitrary", "arbitrary")),
    )(mn, poi, demo, image, W_poi, b_poi.reshape(1, -1), W1,
      b1.reshape(1, -1), W2, b2.reshape(1, -1), W_img, b_img.reshape(1, -1))


def kernel(poi, demo, image, mob_adj, global_indices, ebds,
           W_poi, b_poi, W1, b1, W2, b2, W_img, b_img):
    eg = _sc_gather_rows_kernel()(ebds, global_indices)
    y = _propagate(mob_adj, ebds)
    mob_ebd, mob_n = _mob_embed(global_indices, mob_adj, y, eg)
    l_mp, l_pm, l_md, l_dm, l_mi, l_im = _logits(
        mob_n, poi, demo, image, W_poi, b_poi, W1, b1, W2, b2, W_img, b_img)
    return (l_mp, l_pm, l_md, l_dm, l_mi, l_im, mob_ebd)
